# baseline (speedup 1.0000x reference)
import jax
import jax.numpy as jnp
from jax import lax
from jax.experimental import pallas as pl
from jax.experimental.pallas import tpu as pltpu
from jax.experimental.pallas import tpu_sc as plsc

B = 16384
D = 32
NCONT = 13
NPXPY = 2
W1 = NCONT + NPXPY + 2 * D
NC, NS = 2, 16
NW = NC * NS
BPW = B // NW
C = 32
NCH = BPW // C


def _fire_slabs(tables, r0, s_v, sg):
    for t, (e_hbm, i_v) in enumerate(tables):
        for m in range(C // 16):
            vj = lax.shift_right_logical(i_v[pl.ds(r0 + m * 16, 16)], 3)
            for l in range(16):
                g = vj[l]
                pltpu.async_copy(e_hbm.at[pl.ds(g, 1)],
                                 s_v.at[t, pl.ds(m * 16 + l, 1)], sg)


def _drain_slabs(n, e_hbm, s_v, sg):
    def drain(j, _):
        pltpu.make_async_copy(e_hbm.at[pl.ds(0, 1)], s_v.at[0, pl.ds(0, 1)], sg).wait()
        return ()
    lax.fori_loop(0, n, drain, ())


def _body1(cont_hbm, pxpy_hbm, c0_hbm, c1_hbm, e0_hbm, e1_hbm,
           out_hbm, i0_v, i1_v, c_v, p_v, s_v, o_v, sg, sc, sp):
    wid = lax.axis_index("s") * NC + lax.axis_index("c")
    base = wid * BPW
    lane = lax.iota(jnp.int32, 16)

    pltpu.sync_copy(c0_hbm.at[pl.ds(base, BPW)], i0_v)
    pltpu.sync_copy(c1_hbm.at[pl.ds(base, BPW)], i1_v)

    def chunk(k, _):
        r0 = k * C
        cpc = pltpu.async_copy(cont_hbm.at[pl.ds(base + r0, C)], c_v, sc)
        cpp = pltpu.async_copy(pxpy_hbm.at[pl.ds(base + r0, C)], p_v, sp)
        _fire_slabs([(e0_hbm, i0_v), (e1_hbm, i1_v)], r0, s_v, sg)
        _drain_slabs(2 * C, e0_hbm, s_v, sg)
        cpc.wait()
        cpp.wait()

        def arow(j, _):
            jj = jnp.full((16,), j, jnp.int32)
            rr = jj + r0
            u0 = lax.bitwise_and(plsc.load_gather(i0_v, [rr]), 7)
            u1 = lax.bitwise_and(plsc.load_gather(i1_v, [rr]), 7)
            wcont = plsc.load_gather(c_v, [jj, jnp.minimum(lane, NCONT - 1)])
            wpx = plsc.load_gather(p_v, [jj, jnp.clip(lane - NCONT, 0, NPXPY - 1)])

            def epiece(t, u, col):
                tt = jnp.full((16,), t, jnp.int32)
                return plsc.load_gather(s_v, [tt, jj, u, jnp.clip(col, 0, D - 1)])

            t0 = jnp.where(lane < NCONT, wcont,
                           jnp.where(lane < NCONT + NPXPY, wpx, epiece(0, u0, lane - 15)))
            t1 = epiece(0, u0, lane + 1)
            t2 = jnp.where(lane < 15, epiece(0, u0, lane + 17), epiece(1, u1, lane - 15))
            t3 = epiece(1, u1, lane + 1)
            t4 = epiece(1, u1, lane + 17)
            plsc.store_scatter(o_v, [jj, lane], t0)
            plsc.store_scatter(o_v, [jj, lane + 16], t1)
            plsc.store_scatter(o_v, [jj, lane + 32], t2)
            plsc.store_scatter(o_v, [jj, lane + 48], t3)
            plsc.store_scatter(o_v, [jj, jnp.minimum(lane + 64, W1 - 1)], t4,
                               mask=lane < 15)
            return ()
        lax.fori_loop(0, C, arow, ())

        pltpu.sync_copy(o_v, out_hbm.at[pl.ds(base + r0, C)])
        return ()
    lax.fori_loop(0, NCH, chunk, ())


def _body2(c2_hbm, e2_hbm, out_hbm, i2_v, s_v, o_v, sg):
    wid = lax.axis_index("s") * NC + lax.axis_index("c")
    base = wid * BPW
    lane = lax.iota(jnp.int32, 16)

    pltpu.sync_copy(c2_hbm.at[pl.ds(base, BPW)], i2_v)

    def chunk(k, _):
        r0 = k * C
        _fire_slabs([(e2_hbm, i2_v)], r0, s_v, sg)
        _drain_slabs(C, e2_hbm, s_v, sg)

        def arow(j, _):
            jj = jnp.full((16,), j, jnp.int32)
            zz = jnp.zeros((16,), jnp.int32)
            u2 = lax.bitwise_and(plsc.load_gather(i2_v, [jj + r0]), 7)
            t0 = plsc.load_gather(s_v, [zz, jj, u2, lane])
            t1 = plsc.load_gather(s_v, [zz, jj, u2, lane + 16])
            plsc.store_scatter(o_v, [jj, lane], t0)
            plsc.store_scatter(o_v, [jj, lane + 16], t1)
            return ()
        lax.fori_loop(0, C, arow, ())

        pltpu.sync_copy(o_v, out_hbm.at[pl.ds(base + r0, C)])
        return ()
    lax.fori_loop(0, NCH, chunk, ())


def kernel(continuous, pxpy, cat0, cat1, cat2, emb0, emb1, emb2):
    e0_3 = emb0.reshape(emb0.shape[0] // 8, 8, D)
    e1_3 = emb1.reshape(emb1.shape[0] // 8, 8, D)
    e2_3 = emb2.reshape(emb2.shape[0] // 8, 8, D)
    mesh = plsc.VectorSubcoreMesh(core_axis_name="c", subcore_axis_name="s")
    cp = pltpu.CompilerParams(needs_layout_passes=False)
    run1 = pl.kernel(
        _body1,
        out_type=jax.ShapeDtypeStruct((B, W1), jnp.float32),
        mesh=mesh,
        scratch_types=[
            pltpu.VMEM((BPW,), jnp.int32),
            pltpu.VMEM((BPW,), jnp.int32),
            pltpu.VMEM((C, NCONT), jnp.float32),
            pltpu.VMEM((C, NPXPY), jnp.float32),
            pltpu.VMEM((2, C, 8, D), jnp.float32),
            pltpu.VMEM((C, W1), jnp.float32),
            pltpu.SemaphoreType.DMA,
            pltpu.SemaphoreType.DMA,
            pltpu.SemaphoreType.DMA,
        ],
        compiler_params=cp,
    )
    run2 = pl.kernel(
        _body2,
        out_type=jax.ShapeDtypeStruct((B, D), jnp.float32),
        mesh=mesh,
        scratch_types=[
            pltpu.VMEM((BPW,), jnp.int32),
            pltpu.VMEM((1, C, 8, D), jnp.float32),
            pltpu.VMEM((C, D), jnp.float32),
            pltpu.SemaphoreType.DMA,
        ],
        compiler_params=cp,
    )
    out1 = run1(continuous, pxpy, cat0, cat1, emb0.reshape(emb0.shape[0] // 8, 8, D), e1_3)
    out2 = run2(cat2, e2_3)
    return jnp.concatenate([out1, out2], axis=-1)


# speedup vs baseline: 1.0007x; 1.0007x over previous
import jax
import jax.numpy as jnp
from jax import lax
from jax.experimental import pallas as pl
from jax.experimental.pallas import tpu as pltpu
from jax.experimental.pallas import tpu_sc as plsc
from jax.experimental import layout as jlayout

B = 16384
D = 32
NCONT = 13
NPXPY = 2
W1 = NCONT + NPXPY + 2 * D
NC, NS = 2, 16
NW = NC * NS
BPW = B // NW
C = 32
NCH = BPW // C


def _fire_slabs(tables, r0, s_v, sg):
    for t, (e_hbm, i_v) in enumerate(tables):
        for m in range(C // 16):
            vj = lax.shift_right_logical(i_v[pl.ds(r0 + m * 16, 16)], 3)
            for l in range(16):
                g = vj[l]
                pltpu.async_copy(e_hbm.at[pl.ds(g, 1)],
                                 s_v.at[t, pl.ds(m * 16 + l, 1)], sg)


def _drain_slabs(n, e_hbm, s_v, sg):
    def drain(j, _):
        pltpu.make_async_copy(e_hbm.at[pl.ds(0, 1)], s_v.at[0, pl.ds(0, 1)], sg).wait()
        return ()
    lax.fori_loop(0, n, drain, ())


def _body1(cont_hbm, pxpy_hbm, c0_hbm, c1_hbm, e0_hbm, e1_hbm,
           out_hbm, i0_v, i1_v, c_v, p_v, s_v, o_v, sg, sc, sp):
    wid = lax.axis_index("s") * NC + lax.axis_index("c")
    base = wid * BPW
    lane = lax.iota(jnp.int32, 16)

    pltpu.sync_copy(c0_hbm.at[pl.ds(base, BPW)], i0_v)
    pltpu.sync_copy(c1_hbm.at[pl.ds(base, BPW)], i1_v)

    def chunk(k, _):
        r0 = k * C
        cpc = pltpu.async_copy(cont_hbm.at[pl.ds(base + r0, C)], c_v, sc)
        cpp = pltpu.async_copy(pxpy_hbm.at[pl.ds(base + r0, C)], p_v, sp)
        _fire_slabs([(e0_hbm, i0_v), (e1_hbm, i1_v)], r0, s_v, sg)
        _drain_slabs(2 * C, e0_hbm, s_v, sg)
        cpc.wait()
        cpp.wait()

        def arow(j, _):
            jj = jnp.full((16,), j, jnp.int32)
            rr = jj + r0
            u0 = lax.bitwise_and(plsc.load_gather(i0_v, [rr]), 7)
            u1 = lax.bitwise_and(plsc.load_gather(i1_v, [rr]), 7)
            wcont = plsc.load_gather(c_v, [jj, jnp.minimum(lane, NCONT - 1)])
            wpx = plsc.load_gather(p_v, [jj, jnp.clip(lane - NCONT, 0, NPXPY - 1)])

            def epiece(t, u, col):
                tt = jnp.full((16,), t, jnp.int32)
                return plsc.load_gather(s_v, [tt, jj, u, jnp.clip(col, 0, D - 1)])

            t0 = jnp.where(lane < NCONT, wcont,
                           jnp.where(lane < NCONT + NPXPY, wpx, epiece(0, u0, lane - 15)))
            t1 = epiece(0, u0, lane + 1)
            t2 = jnp.where(lane < 15, epiece(0, u0, lane + 17), epiece(1, u1, lane - 15))
            t3 = epiece(1, u1, lane + 1)
            t4 = epiece(1, u1, lane + 17)
            plsc.store_scatter(o_v, [jj, lane], t0)
            plsc.store_scatter(o_v, [jj, lane + 16], t1)
            plsc.store_scatter(o_v, [jj, lane + 32], t2)
            plsc.store_scatter(o_v, [jj, lane + 48], t3)
            plsc.store_scatter(o_v, [jj, jnp.minimum(lane + 64, W1 - 1)], t4,
                               mask=lane < 15)
            return ()
        lax.fori_loop(0, C, arow, ())

        pltpu.sync_copy(o_v, out_hbm.at[pl.ds(base + r0, C)])
        return ()
    lax.fori_loop(0, NCH, chunk, ())


def _body2(c2_hbm, e2_hbm, out_hbm, i2_v, s_v, o_v, sg):
    wid = lax.axis_index("s") * NC + lax.axis_index("c")
    base = wid * BPW
    lane = lax.iota(jnp.int32, 16)

    pltpu.sync_copy(c2_hbm.at[pl.ds(base, BPW)], i2_v)

    def chunk(k, _):
        r0 = k * C
        _fire_slabs([(e2_hbm, i2_v)], r0, s_v, sg)
        _drain_slabs(C, e2_hbm, s_v, sg)

        def arow(j, _):
            jj = jnp.full((16,), j, jnp.int32)
            zz = jnp.zeros((16,), jnp.int32)
            u2 = lax.bitwise_and(plsc.load_gather(i2_v, [jj + r0]), 7)
            t0 = plsc.load_gather(s_v, [zz, jj, u2, lane])
            t1 = plsc.load_gather(s_v, [zz, jj, u2, lane + 16])
            plsc.store_scatter(o_v, [jj, lane], t0)
            plsc.store_scatter(o_v, [jj, lane + 16], t1)
            return ()
        lax.fori_loop(0, C, arow, ())

        pltpu.sync_copy(o_v, out_hbm.at[pl.ds(base + r0, C)])
        return ()
    lax.fori_loop(0, NCH, chunk, ())


def kernel(continuous, pxpy, cat0, cat1, cat2, emb0, emb1, emb2):
    e2t = lax.optimization_barrier(jnp.swapaxes(emb2, 0, 1))
    emb2 = jlayout.with_layout_constraint(
        jnp.swapaxes(e2t, 0, 1), jlayout.Layout((0, 1), ((8, 128),)))
    e0_3 = emb0.reshape(emb0.shape[0] // 8, 8, D)
    e1_3 = emb1.reshape(emb1.shape[0] // 8, 8, D)
    e2_3 = emb2.reshape(emb2.shape[0] // 8, 8, D)
    mesh = plsc.VectorSubcoreMesh(core_axis_name="c", subcore_axis_name="s")
    cp = pltpu.CompilerParams(needs_layout_passes=False)
    run1 = pl.kernel(
        _body1,
        out_type=jax.ShapeDtypeStruct((B, W1), jnp.float32),
        mesh=mesh,
        scratch_types=[
            pltpu.VMEM((BPW,), jnp.int32),
            pltpu.VMEM((BPW,), jnp.int32),
            pltpu.VMEM((C, NCONT), jnp.float32),
            pltpu.VMEM((C, NPXPY), jnp.float32),
            pltpu.VMEM((2, C, 8, D), jnp.float32),
            pltpu.VMEM((C, W1), jnp.float32),
            pltpu.SemaphoreType.DMA,
            pltpu.SemaphoreType.DMA,
            pltpu.SemaphoreType.DMA,
        ],
        compiler_params=cp,
    )
    run2 = pl.kernel(
        _body2,
        out_type=jax.ShapeDtypeStruct((B, D), jnp.float32),
        mesh=mesh,
        scratch_types=[
            pltpu.VMEM((BPW,), jnp.int32),
            pltpu.VMEM((1, C, 8, D), jnp.float32),
            pltpu.VMEM((C, D), jnp.float32),
            pltpu.SemaphoreType.DMA,
        ],
        compiler_params=cp,
    )
    out1 = run1(continuous, pxpy, cat0, cat1, emb0.reshape(emb0.shape[0] // 8, 8, D), e1_3)
    out2 = run2(cat2, e2_3)
    return jnp.concatenate([out1, out2], axis=-1)


# speedup vs baseline: 1.1351x; 1.1343x over previous
import jax
import jax.numpy as jnp
from jax import lax
from jax.experimental import pallas as pl
from jax.experimental.pallas import tpu as pltpu
from jax.experimental.pallas import tpu_sc as plsc

B = 16384
D = 32
NCONT = 13
NPXPY = 2
OUT_W = NCONT + NPXPY + 3 * D
NC, NS = 2, 16
NW = NC * NS
BPW = B // NW
C = 32
NCH = BPW // C


def _body(cont_hbm, pxpy_hbm, c0_hbm, c1_hbm, c2_hbm, e0_hbm, e1_hbm, e2_hbm,
          out_hbm,
          i0_v, i1_v, i2_v, c_v, p_v, s_v, o_v,
          sg, sc, sp):
    wid = lax.axis_index("s") * NC + lax.axis_index("c")
    base = wid * BPW
    lane = lax.iota(jnp.int32, 16)

    pltpu.sync_copy(c0_hbm.at[pl.ds(base, BPW)], i0_v)
    pltpu.sync_copy(c1_hbm.at[pl.ds(base, BPW)], i1_v)
    pltpu.sync_copy(c2_hbm.at[pl.ds(base, BPW)], i2_v)

    def chunk(k, _):
        r0 = k * C
        cpc = pltpu.async_copy(cont_hbm.at[pl.ds(base + r0, C)], c_v, sc)
        cpp = pltpu.async_copy(pxpy_hbm.at[pl.ds(base + r0, C)], p_v, sp)

        def fire(t, e_hbm, i_v):
            for m in range(C // 16):
                vj = lax.shift_right_logical(i_v[pl.ds(r0 + m * 16, 16)], 3)
                for l in range(16):
                    g = vj[l]
                    pltpu.async_copy(e_hbm.at[pl.ds(g, 1)],
                                     s_v.at[t, pl.ds(m * 16 + l, 1)], sg)
        fire(0, e0_hbm, i0_v)
        fire(1, e1_hbm, i1_v)
        fire(2, e2_hbm, i2_v)

        def drain(j, _):
            pltpu.make_async_copy(e0_hbm.at[pl.ds(0, 1)], s_v.at[0, pl.ds(0, 1)], sg).wait()
            pltpu.make_async_copy(e0_hbm.at[pl.ds(0, 1)], s_v.at[0, pl.ds(0, 1)], sg).wait()
            pltpu.make_async_copy(e0_hbm.at[pl.ds(0, 1)], s_v.at[0, pl.ds(0, 1)], sg).wait()
            return ()
        lax.fori_loop(0, C, drain, ())
        cpc.wait()
        cpp.wait()

        def arow(j, _):
            jj = jnp.full((16,), j, jnp.int32)
            rr = jj + r0
            u0 = lax.bitwise_and(plsc.load_gather(i0_v, [rr]), 7)
            u1 = lax.bitwise_and(plsc.load_gather(i1_v, [rr]), 7)
            u2 = lax.bitwise_and(plsc.load_gather(i2_v, [rr]), 7)
            wcont = plsc.load_gather(c_v, [jj, jnp.minimum(lane, NCONT - 1)])
            wpx = plsc.load_gather(p_v, [jj, jnp.clip(lane - NCONT, 0, NPXPY - 1)])

            def epiece(t, u, col):
                tt = jnp.full((16,), t, jnp.int32)
                return plsc.load_gather(s_v, [tt, jj, u, jnp.clip(col, 0, D - 1)])

            t0 = jnp.where(lane < NCONT, wcont,
                           jnp.where(lane < NCONT + NPXPY, wpx, epiece(0, u0, lane - 15)))
            t1 = epiece(0, u0, lane + 1)
            t2 = jnp.where(lane < 15, epiece(0, u0, lane + 17), epiece(1, u1, lane - 15))
            t3 = epiece(1, u1, lane + 1)
            t4 = jnp.where(lane < 15, epiece(1, u1, lane + 17), epiece(2, u2, lane - 15))
            t5 = epiece(2, u2, lane + 1)
            t6 = epiece(2, u2, lane + 17)
            plsc.store_scatter(o_v, [jj, lane], t0)
            plsc.store_scatter(o_v, [jj, lane + 16], t1)
            plsc.store_scatter(o_v, [jj, lane + 32], t2)
            plsc.store_scatter(o_v, [jj, lane + 48], t3)
            plsc.store_scatter(o_v, [jj, lane + 64], t4)
            plsc.store_scatter(o_v, [jj, lane + 80], t5)
            plsc.store_scatter(o_v, [jj, jnp.minimum(lane + 96, OUT_W - 1)], t6,
                               mask=lane < 15)
            return ()
        lax.fori_loop(0, C, arow, ())

        pltpu.sync_copy(o_v, out_hbm.at[pl.ds(base + r0, C)])
        return ()
    lax.fori_loop(0, NCH, chunk, ())


def kernel(continuous, pxpy, cat0, cat1, cat2, emb0, emb1, emb2):
    e0_3 = emb0.reshape(emb0.shape[0] // 8, 8, D)
    e1_3 = emb1.reshape(emb1.shape[0] // 8, 8, D)
    e2_3 = emb2.reshape(emb2.shape[0] // 8, 8, D)
    mesh = plsc.VectorSubcoreMesh(core_axis_name="c", subcore_axis_name="s")
    run = pl.kernel(
        _body,
        out_type=jax.ShapeDtypeStruct((B, OUT_W), jnp.float32),
        mesh=mesh,
        scratch_types=[
            pltpu.VMEM((BPW,), jnp.int32),
            pltpu.VMEM((BPW,), jnp.int32),
            pltpu.VMEM((BPW,), jnp.int32),
            pltpu.VMEM((C, NCONT), jnp.float32),
            pltpu.VMEM((C, NPXPY), jnp.float32),
            pltpu.VMEM((3, C, 8, D), jnp.float32),
            pltpu.VMEM((C, OUT_W), jnp.float32),
            pltpu.SemaphoreType.DMA,
            pltpu.SemaphoreType.DMA,
            pltpu.SemaphoreType.DMA,
        ],
        compiler_params=pltpu.CompilerParams(needs_layout_passes=False),
    )
    return run(continuous, pxpy, cat0, cat1, cat2, e0_3, e1_3, e2_3)
